# Initial kernel scaffold; baseline (speedup 1.0000x reference)
import jax
import jax.numpy as jnp
from jax.experimental import pallas as pl


def kernel(h, op_id, expert_key):
    raise NotImplementedError("write your pallas kernel here")


# speedup vs baseline: 3.2350x; 3.2350x over previous
import functools

import jax
import jax.numpy as jnp
from jax.experimental import pallas as pl

N_BUCKET = 64
EPB = 8
NKEYS = N_BUCKET * EPB


def _router_block(h_ref, b_ref, keys_ref, gid_ref, w_ref):
    h = h_ref[...]
    keys = keys_ref[...]

    hn = h * (1.0 / jnp.maximum(jnp.sqrt(jnp.sum(h * h, axis=1, keepdims=True)), 1e-12))
    kn = keys * (1.0 / jnp.maximum(jnp.sqrt(jnp.sum(keys * keys, axis=1, keepdims=True)), 1e-12))

    scores = jax.lax.dot_general(
        hn, kn, (((1,), (1,)), ((), ())),
        preferred_element_type=jnp.float32,
        precision=jax.lax.Precision.HIGHEST,
    )

    Tb = h.shape[0]
    lane = jax.lax.broadcasted_iota(jnp.int32, (Tb, NKEYS), 1)
    base = b_ref[...] * EPB
    mask = (lane >= base) & (lane < base + EPB)

    neg = jnp.float32(-1e30)
    s = jnp.where(mask, scores, neg)

    m = jnp.max(s, axis=1, keepdims=True)
    e = jnp.where(mask, jnp.exp(scores - m), 0.0)
    S = jnp.sum(e, axis=1, keepdims=True)

    big = jnp.int32(NKEYS)
    i1 = jnp.min(jnp.where(s == m, lane, big), axis=1, keepdims=True)
    p1 = 1.0 / S

    s2 = jnp.where(lane == i1, neg, s)
    m2 = jnp.max(s2, axis=1, keepdims=True)
    i2 = jnp.min(jnp.where(s2 == m2, lane, big), axis=1, keepdims=True)
    p2 = jnp.exp(m2 - m) / S

    denom = p1 + p2 + 1e-9
    gid_ref[...] = jnp.concatenate([i1, i2], axis=1)
    w_ref[...] = jnp.concatenate([p1 / denom, p2 / denom], axis=1).astype(jnp.float32)


@jax.jit
def _route(h2, b2, keys2):
    T, C = h2.shape
    Tb = 512
    grid = (T // Tb,)
    gid, w = pl.pallas_call(
        _router_block,
        grid=grid,
        in_specs=[
            pl.BlockSpec((Tb, C), lambda i: (i, 0)),
            pl.BlockSpec((Tb, 1), lambda i: (i, 0)),
            pl.BlockSpec((NKEYS, C), lambda i: (0, 0)),
        ],
        out_specs=[
            pl.BlockSpec((Tb, 2), lambda i: (i, 0)),
            pl.BlockSpec((Tb, 2), lambda i: (i, 0)),
        ],
        out_shape=[
            jax.ShapeDtypeStruct((T, 2), jnp.int32),
            jax.ShapeDtypeStruct((T, 2), jnp.float32),
        ],
    )(h2, b2, keys2)
    return gid, w


def kernel(h, op_id, expert_key):
    B, T, C = h.shape
    h2 = h.reshape(B * T, C)
    b2 = jnp.clip(op_id, 0, N_BUCKET - 1).astype(jnp.int32).reshape(B * T, 1)
    keys2 = expert_key.reshape(NKEYS, C)
    gid, w = _route(h2, b2, keys2)
    return gid.reshape(B, T, 2), w.reshape(B, T, 2)


# speedup vs baseline: 6.9875x; 2.1599x over previous
import jax
import jax.numpy as jnp
from jax.experimental import pallas as pl
from jax.experimental.pallas import tpu as pltpu

N_BUCKET = 64
EPB = 8
NKEYS = N_BUCKET * EPB


def _router_block(h_ref, b_ref, keys_ref, gid_ref, w_ref, kn_ref):
    @pl.when(pl.program_id(0) == 0)
    def _normalize_keys():
        keys = keys_ref[...]
        norm = jnp.sqrt(jnp.sum(keys * keys, axis=1, keepdims=True))
        kn_ref[...] = keys * (1.0 / jnp.maximum(norm, 1e-12))

    h = h_ref[...]
    rh = 1.0 / jnp.maximum(jnp.sqrt(jnp.sum(h * h, axis=1, keepdims=True)), 1e-12)

    scores = jax.lax.dot_general(
        h, kn_ref[...], (((1,), (1,)), ((), ())),
        preferred_element_type=jnp.float32,
        precision=jax.lax.Precision.DEFAULT,
    ) * rh

    Tb = h.shape[0]
    lane = jax.lax.broadcasted_iota(jnp.int32, (Tb, NKEYS), 1)
    mask = (lane // EPB) == b_ref[...]

    neg = jnp.float32(-1e30)
    s = jnp.where(mask, scores, neg)

    m = jnp.max(s, axis=1, keepdims=True)
    S = jnp.sum(jnp.exp(s - m), axis=1, keepdims=True)

    big = jnp.int32(NKEYS)
    i1 = jnp.min(jnp.where(s == m, lane, big), axis=1, keepdims=True)
    p1 = 1.0 / S

    s2 = jnp.where(lane == i1, neg, s)
    m2 = jnp.max(s2, axis=1, keepdims=True)
    i2 = jnp.min(jnp.where(s2 == m2, lane, big), axis=1, keepdims=True)
    p2 = jnp.exp(m2 - m) * p1

    denom = p1 + p2 + 1e-9
    gid_ref[...] = jnp.concatenate([i1, i2], axis=1)
    w_ref[...] = jnp.concatenate([p1 / denom, p2 / denom], axis=1).astype(jnp.float32)


@jax.jit
def _route(h2, b2, keys2):
    T, C = h2.shape
    Tb = 512
    grid = (T // Tb,)
    gid, w = pl.pallas_call(
        _router_block,
        grid=grid,
        in_specs=[
            pl.BlockSpec((Tb, C), lambda i: (i, 0)),
            pl.BlockSpec((Tb, 1), lambda i: (i, 0)),
            pl.BlockSpec((NKEYS, C), lambda i: (0, 0)),
        ],
        out_specs=[
            pl.BlockSpec((Tb, 2), lambda i: (i, 0)),
            pl.BlockSpec((Tb, 2), lambda i: (i, 0)),
        ],
        out_shape=[
            jax.ShapeDtypeStruct((T, 2), jnp.int32),
            jax.ShapeDtypeStruct((T, 2), jnp.float32),
        ],
        scratch_shapes=[pltpu.VMEM((NKEYS, C), jnp.float32)],
    )(h2, b2, keys2)
    return gid, w


def kernel(h, op_id, expert_key):
    B, T, C = h.shape
    h2 = h.reshape(B * T, C)
    b2 = jnp.clip(op_id, 0, N_BUCKET - 1).astype(jnp.int32).reshape(B * T, 1)
    keys2 = expert_key.reshape(NKEYS, C)
    gid, w = _route(h2, b2, keys2)
    return gid.reshape(B, T, 2), w.reshape(B, T, 2)
